# Initial kernel scaffold; baseline (speedup 1.0000x reference)
import jax
import jax.numpy as jnp
from jax.experimental import pallas as pl


def kernel(xn, xe, edge_index, KNopen, KEopen, KNclose, KEclose, KE1, KE2, KN):
    raise NotImplementedError("write your pallas kernel here")


# speedup vs baseline: 4.0224x; 4.0224x over previous
import functools

import jax
import jax.numpy as jnp
from jax import lax
from jax.experimental import pallas as pl
from jax.experimental.pallas import tpu as pltpu
from jax.experimental.pallas import tpu_sc as plsc

_F32 = jnp.float32
_NC = 2
_NS = 16
_NW = _NC * _NS
_CHUNK = 128
_EDGE_BLK = 2000


def _sc_mesh():
    return plsc.VectorSubcoreMesh(
        core_axis_name="c", subcore_axis_name="s",
        num_cores=_NC, num_subcores=_NS)


def _sc_gather(u, v, ii, jj):
    e = ii.shape[0]
    nchunks = e // _CHUNK
    per_w = -(-nchunks // _NW)
    d = u.shape[1]

    @functools.partial(
        pl.kernel,
        out_type=[jax.ShapeDtypeStruct((e, d), _F32),
                  jax.ShapeDtypeStruct((e, d), _F32)],
        mesh=_sc_mesh(),
        scratch_types=[
            pltpu.VMEM((_CHUNK,), jnp.int32),
            pltpu.VMEM((_CHUNK,), jnp.int32),
            pltpu.VMEM((_CHUNK, d), _F32),
            pltpu.VMEM((_CHUNK, d), _F32),
            pltpu.SemaphoreType.DMA,
            pltpu.SemaphoreType.DMA,
        ],
    )
    def run(u_hbm, v_hbm, ii_hbm, jj_hbm, g1_hbm, g2_hbm,
            i1_v, i2_v, r1_v, r2_v, s1, s2):
        wid = lax.axis_index("s") * _NC + lax.axis_index("c")

        def body(t, carry):
            c = wid + t * _NW

            @pl.when(c < nchunks)
            def _():
                base = c * _CHUNK
                pltpu.sync_copy(ii_hbm.at[pl.ds(base, _CHUNK)], i1_v)
                pltpu.sync_copy(jj_hbm.at[pl.ds(base, _CHUNK)], i2_v)
                cp1 = pltpu.async_copy(u_hbm.at[i1_v], r1_v, s1)
                cp2 = pltpu.async_copy(v_hbm.at[i2_v], r2_v, s2)
                cp1.wait()
                cp2.wait()
                pltpu.sync_copy(r1_v, g1_hbm.at[pl.ds(base, _CHUNK)])
                pltpu.sync_copy(r2_v, g2_hbm.at[pl.ds(base, _CHUNK)])
            return carry

        lax.fori_loop(0, per_w, body, 0)

    return run(u, v, ii, jj)


def _sc_scatter(y, ii, jj, zeros_n):
    e, d = y.shape
    n = zeros_n.shape[0]
    nchunks = e // _CHUNK
    per_t = -(-nchunks // _NS)
    rows_pt = (n // _NS) // 8 * 8
    rem = n - rows_pt * _NS

    @functools.partial(
        pl.kernel,
        out_type=[jax.ShapeDtypeStruct((n, d), _F32),
                  jax.ShapeDtypeStruct((n, d), _F32)],
        mesh=_sc_mesh(),
        scratch_types=[
            pltpu.VMEM((_CHUNK,), jnp.int32),
            pltpu.VMEM((_CHUNK, d), _F32),
            pltpu.VMEM_SHARED((n, d), _F32),
        ],
    )
    def run(y_hbm, ii_hbm, jj_hbm, z_hbm, si_hbm, sj_hbm,
            idx_v, rows_v, acc_sh):
        cid = lax.axis_index("c")
        sid = lax.axis_index("s")
        r0 = sid * rows_pt
        pltpu.sync_copy(z_hbm.at[pl.ds(r0, rows_pt)],
                        acc_sh.at[pl.ds(r0, rows_pt)])
        if rem:
            @pl.when(sid == 0)
            def _():
                pltpu.sync_copy(z_hbm.at[pl.ds(rows_pt * _NS, rem)],
                                acc_sh.at[pl.ds(rows_pt * _NS, rem)])
        plsc.subcore_barrier()

        def accumulate(idx_hbm):
            def body(t, carry):
                c = sid + t * _NS

                @pl.when(c < nchunks)
                def _():
                    base = c * _CHUNK
                    pltpu.sync_copy(idx_hbm.at[pl.ds(base, _CHUNK)], idx_v)
                    pltpu.sync_copy(y_hbm.at[pl.ds(base, _CHUNK)], rows_v)
                    pltpu.sync_copy(rows_v, acc_sh.at[idx_v], add=True)
                return carry

            lax.fori_loop(0, per_t, body, 0)

        @pl.when(cid == 0)
        def _():
            accumulate(ii_hbm)

        @pl.when(cid == 1)
        def _():
            accumulate(jj_hbm)

        plsc.subcore_barrier()

        @pl.when(cid == 0)
        def _():
            pltpu.sync_copy(acc_sh.at[pl.ds(r0, rows_pt)],
                            si_hbm.at[pl.ds(r0, rows_pt)])
            if rem:
                @pl.when(sid == 0)
                def _():
                    pltpu.sync_copy(acc_sh.at[pl.ds(rows_pt * _NS, rem)],
                                    si_hbm.at[pl.ds(rows_pt * _NS, rem)])

        @pl.when(cid == 1)
        def _():
            pltpu.sync_copy(acc_sh.at[pl.ds(r0, rows_pt)],
                            sj_hbm.at[pl.ds(r0, rows_pt)])
            if rem:
                @pl.when(sid == 0)
                def _():
                    pltpu.sync_copy(acc_sh.at[pl.ds(rows_pt * _NS, rem)],
                                    sj_hbm.at[pl.ds(rows_pt * _NS, rem)])

    return run(y, ii, jj, zeros_n)


def _tv_relu(t):
    t = t - jnp.mean(t, axis=1, keepdims=True)
    t = t / jnp.sqrt(jnp.sum(t * t, axis=1, keepdims=True) + 1e-3)
    return jnp.maximum(t, 0.0)


def _node_prep_body(x0_ref, wo_ref, wi_ref, wj_ref, xn_ref, u_ref, v_ref):
    xn = jnp.dot(x0_ref[...], wo_ref[...], preferred_element_type=_F32)
    xn_ref[...] = xn
    u_ref[...] = jnp.dot(xn, wi_ref[...], preferred_element_type=_F32)
    v_ref[...] = jnp.dot(xn, wj_ref[...], preferred_element_type=_F32)


def _node_prep(x0, wo_t, wi_t, wj_t):
    n, d = x0.shape[0], wo_t.shape[1]
    return pl.pallas_call(
        _node_prep_body,
        out_shape=[jax.ShapeDtypeStruct((n, d), _F32)] * 3,
    )(x0, wo_t, wi_t, wj_t)


def _edge1_body(g1_ref, g2_ref, xe0_ref, wo_ref, wa_ref, wk2_ref,
                xec_ref, xe1_ref):
    xeo = jnp.dot(xe0_ref[...], wo_ref[...], preferred_element_type=_F32)
    t = g1_ref[...] + g2_ref[...] + jnp.dot(
        xeo, wa_ref[...], preferred_element_type=_F32)
    xec = _tv_relu(t)
    xec_ref[...] = xec
    xe1_ref[...] = xeo + jnp.dot(xec, wk2_ref[...],
                                 preferred_element_type=_F32)


def _edge_layer1(g1, g2, xe0, wo_t, wa_t, wk2_t):
    e, d = g1.shape
    d0 = xe0.shape[1]
    grid = e // _EDGE_BLK
    blk = lambda w: pl.BlockSpec((_EDGE_BLK, w), lambda i: (i, 0))
    wspec = lambda a: pl.BlockSpec(a.shape, lambda i: (0, 0))
    return pl.pallas_call(
        _edge1_body,
        grid=(grid,),
        in_specs=[blk(d), blk(d), blk(d0),
                  wspec(wo_t), wspec(wa_t), wspec(wk2_t)],
        out_specs=[blk(d), blk(d)],
        out_shape=[jax.ShapeDtypeStruct((e, d), _F32)] * 2,
    )(g1, g2, xe0, wo_t, wa_t, wk2_t)


def _edge2_body(g1_ref, g2_ref, xe_ref, wa_ref, wk2_ref, wc_ref,
                xec_ref, xcl_ref):
    xe = xe_ref[...]
    t = g1_ref[...] + g2_ref[...] + jnp.dot(
        xe, wa_ref[...], preferred_element_type=_F32)
    xec = _tv_relu(t)
    xec_ref[...] = xec
    xe2 = xe + jnp.dot(xec, wk2_ref[...], preferred_element_type=_F32)
    xcl_ref[...] = jnp.dot(xe2, wc_ref[...], preferred_element_type=_F32)


def _edge_layer2(g1, g2, xe, wa_t, wk2_t, wc_t):
    e, d = g1.shape
    dc = wc_t.shape[1]
    grid = e // _EDGE_BLK
    blk = lambda w: pl.BlockSpec((_EDGE_BLK, w), lambda i: (i, 0))
    wspec = lambda a: pl.BlockSpec(a.shape, lambda i: (0, 0))
    return pl.pallas_call(
        _edge2_body,
        grid=(grid,),
        in_specs=[blk(d), blk(d), blk(d),
                  wspec(wa_t), wspec(wk2_t), wspec(wc_t)],
        out_specs=[blk(d), blk(dc)],
        out_shape=[jax.ShapeDtypeStruct((e, d), _F32),
                   jax.ShapeDtypeStruct((e, dc), _F32)],
    )(g1, g2, xe, wa_t, wk2_t, wc_t)


def _node_upd1_body(xn_ref, si_ref, sj_ref, wb_ref, wp_ref, wq_ref,
                    wi_ref, wj_ref, xn1_ref, u_ref, v_ref):
    xn = xn_ref[...]
    xn1 = (xn + jnp.dot(xn, wb_ref[...], preferred_element_type=_F32)
           + jnp.dot(si_ref[...], wp_ref[...], preferred_element_type=_F32)
           + jnp.dot(sj_ref[...], wq_ref[...], preferred_element_type=_F32))
    xn1_ref[...] = xn1
    u_ref[...] = jnp.dot(xn1, wi_ref[...], preferred_element_type=_F32)
    v_ref[...] = jnp.dot(xn1, wj_ref[...], preferred_element_type=_F32)


def _node_update1(xn, si, sj, wb_t, wp_t, wq_t, wi_t, wj_t):
    n, d = xn.shape
    return pl.pallas_call(
        _node_upd1_body,
        out_shape=[jax.ShapeDtypeStruct((n, d), _F32)] * 3,
    )(xn, si, sj, wb_t, wp_t, wq_t, wi_t, wj_t)


def _node_upd2_body(xn_ref, si_ref, sj_ref, wb_ref, wp_ref, wq_ref,
                    wc_ref, out_ref):
    xn = xn_ref[...]
    xn2 = (xn + jnp.dot(xn, wb_ref[...], preferred_element_type=_F32)
           + jnp.dot(si_ref[...], wp_ref[...], preferred_element_type=_F32)
           + jnp.dot(sj_ref[...], wq_ref[...], preferred_element_type=_F32))
    out_ref[...] = jnp.dot(xn2, wc_ref[...], preferred_element_type=_F32)


def _node_update2(xn, si, sj, wb_t, wp_t, wq_t, wc_t):
    n = xn.shape[0]
    dc = wc_t.shape[1]
    return pl.pallas_call(
        _node_upd2_body,
        out_shape=jax.ShapeDtypeStruct((n, dc), _F32),
    )(xn, si, sj, wb_t, wp_t, wq_t, wc_t)


def kernel(xn, xe, edge_index, KNopen, KEopen, KNclose, KEclose, KE1, KE2, KN):
    h = 0.1
    n = xn.shape[2]
    dn = KNopen.shape[0]

    x0 = xn[0].T
    xe0 = xe[0].T
    ii = edge_index[0]
    jj = edge_index[1]
    zeros_n = jnp.zeros((n, dn), _F32)

    wi_t, wj_t, wa_t, wk2_t = [], [], [], []
    wb_t, wp_t, wq_t = [], [], []
    for l in range(KE1.shape[0]):
        a_int = KE1[l][:, :dn]
        a_xe = KE1[l][:, dn:2 * dn]
        a_grad = KE1[l][:, 2 * dn:]
        wi_t.append((0.5 * a_int + a_grad).T)
        wj_t.append((0.5 * a_int - a_grad).T)
        wa_t.append(a_xe.T)
        wk2_t.append(h * KE2[l].T)
        b_int = KN[l][:, :dn]
        b_xn = KN[l][:, dn:2 * dn]
        b_div = KN[l][:, 2 * dn:]
        wb_t.append(h * b_xn.T)
        wp_t.append(h * (0.5 * b_int + b_div).T)
        wq_t.append(h * (0.5 * b_int - b_div).T)

    xn_o, u, v = _node_prep(x0, KNopen.T, wi_t[0], wj_t[0])
    g1, g2 = _sc_gather(u, v, ii, jj)
    xec, xe1 = _edge_layer1(g1, g2, xe0, KEopen.T, wa_t[0], wk2_t[0])
    si, sj = _sc_scatter(xec, ii, jj, zeros_n)
    xn1, u, v = _node_update1(xn_o, si, sj, wb_t[0], wp_t[0], wq_t[0],
                              wi_t[1], wj_t[1])
    g1, g2 = _sc_gather(u, v, ii, jj)
    xec, xe_cl = _edge_layer2(g1, g2, xe1, wa_t[1], wk2_t[1], KEclose.T)
    si, sj = _sc_scatter(xec, ii, jj, zeros_n)
    xn_cl = _node_update2(xn1, si, sj, wb_t[1], wp_t[1], wq_t[1], KNclose.T)

    return (xn_cl.T[None], xe_cl.T[None])
